# baseline (speedup 1.0000x reference)
import jax
import jax.numpy as jnp
from jax import lax
from jax.experimental import pallas as pl
from jax.experimental.pallas import tpu as pltpu
from jax.experimental.pallas import tpu_sc as plsc

_LANES = 16
_CM = 256
_PAD = 8


def _build_sc_gather(nb, n, c, m):
    info = plsc.get_sparse_core_info()
    num_workers = info.num_cores * info.num_subcores
    assert num_workers % nb == 0
    parts = num_workers // nb
    assert m % parts == 0
    mper = m // parts
    assert mper % _CM == 0
    nch = mper // _CM
    assert nch % 2 == 0 and nch >= 4
    assert c % _LANES == 0
    ncb = c // _LANES

    def body(ft_hbm, idx_hbm, out_hbm, idx_v, g_a, g_b, t_a, t_b,
             s0, s1, t0, t1):
        gbufs = (g_a, g_b)
        tbufs = (t_a, t_b)
        insems = (s0, s1)
        osems = (t0, t1)
        w = lax.axis_index("s") * info.num_cores + lax.axis_index("c")
        b = w // parts
        mb0 = (w % parts) * mper
        pltpu.sync_copy(idx_hbm.at[b].at[pl.ds(mb0, mper)], idx_v)
        table = ft_hbm.at[b]
        lanes = lax.iota(jnp.int32, _LANES)

        def in_start(ch, k):
            pltpu.async_copy(
                table.at[idx_v.at[pl.ds(ch * _CM, _CM)]],
                gbufs[k], insems[k])

        def in_wait(k):
            pltpu.make_async_copy(
                table.at[idx_v.at[pl.ds(0, _CM)]],
                gbufs[k], insems[k]).wait()

        def t_start(ch, cb, p):
            pltpu.async_copy(
                tbufs[p].at[pl.ds(0, _LANES), pl.ds(0, _CM)],
                out_hbm.at[pl.ds(b * c + cb * _LANES, _LANES),
                           pl.ds(mb0 + ch * _CM, _CM)],
                osems[p])

        def t_wait(p):
            pltpu.make_async_copy(
                tbufs[p].at[pl.ds(0, _LANES), pl.ds(0, _CM)],
                out_hbm.at[pl.ds(0, _LANES), pl.ds(0, _CM)],
                osems[p]).wait()

        in_start(0, 0)
        in_start(1, 1)

        def grp(g, carry):
            for k in range(2):
                ch = g * 2 + k
                in_wait(k)
                for cb in range(ncb):
                    p = cb % 2
                    if k == 0 and cb < 2:
                        @pl.when(g > 0)
                        def _():
                            t_wait(p)
                    else:
                        t_wait(p)

                    @plsc.parallel_loop(0, _CM, unroll=8)
                    def _(mm):
                        vals = gbufs[k][mm, pl.ds(cb * _LANES, _LANES)]
                        plsc.store_scatter(
                            tbufs[p], [lanes, jnp.full((_LANES,), mm,
                                                       jnp.int32)], vals)

                    t_start(ch, cb, p)

                @pl.when(ch + 2 < nch)
                def _():
                    in_start(ch + 2, k)
            return carry

        lax.fori_loop(0, nch // 2, grp, 0)
        t_wait(0)
        t_wait(1)

    return pl.kernel(
        body,
        out_type=jax.ShapeDtypeStruct((nb * c, m), jnp.float32),
        mesh=plsc.VectorSubcoreMesh(core_axis_name="c", subcore_axis_name="s"),
        scratch_types=[
            pltpu.VMEM((mper,), jnp.int32),
            pltpu.VMEM((_CM, c), jnp.float32),
            pltpu.VMEM((_CM, c), jnp.float32),
            pltpu.VMEM((_LANES, _CM + _PAD), jnp.float32),
            pltpu.VMEM((_LANES, _CM + _PAD), jnp.float32),
            pltpu.SemaphoreType.DMA,
            pltpu.SemaphoreType.DMA,
            pltpu.SemaphoreType.DMA,
            pltpu.SemaphoreType.DMA,
        ],
        compiler_params=pltpu.CompilerParams(needs_layout_passes=False),
    )


def kernel(features, idx):
    nb, c, n = features.shape
    m = idx.shape[1]
    ft = jnp.swapaxes(features, 1, 2)
    idx32 = idx.astype(jnp.int32)
    gather = _build_sc_gather(nb, n, c, m)
    out2 = gather(ft, idx32)
    return out2.reshape(nb, c, m)


# speedup vs baseline: 2.0711x; 2.0711x over previous
import jax
import jax.numpy as jnp
from jax import lax
from jax.experimental import pallas as pl
from jax.experimental.pallas import tpu as pltpu
from jax.experimental.pallas import tpu_sc as plsc

_CHUNK = 256
_BM = 2048


def _build_sc_gather(boff, nbh, n, c, m):
    info = plsc.get_sparse_core_info()
    num_workers = info.num_cores * info.num_subcores
    assert num_workers % nbh == 0
    parts = num_workers // nbh
    assert m % parts == 0
    mper = m // parts
    assert mper % _CHUNK == 0
    nch = mper // _CHUNK
    assert nch >= 4 and nch % 2 == 0

    def body(ft_hbm, idx_hbm, out_hbm, idx_v, g_a, g_b, s0, s1, t0, t1):
        gbufs = (g_a, g_b)
        insems = (s0, s1)
        osems = (t0, t1)
        w = lax.axis_index("s") * info.num_cores + lax.axis_index("c")
        b = boff + w // parts
        mb0 = (w % parts) * mper
        pltpu.sync_copy(idx_hbm.at[b].at[pl.ds(mb0, mper)], idx_v)
        table = ft_hbm.at[b]
        out_b = out_hbm.at[b - boff]

        def in_start(ch, k):
            pltpu.async_copy(
                table.at[idx_v.at[pl.ds(ch * _CHUNK, _CHUNK)]],
                gbufs[k], insems[k])

        def in_wait(k):
            pltpu.make_async_copy(
                table.at[idx_v.at[pl.ds(0, _CHUNK)]],
                gbufs[k], insems[k]).wait()

        def out_start(ch, k):
            pltpu.async_copy(
                gbufs[k], out_b.at[pl.ds(mb0 + ch * _CHUNK, _CHUNK)],
                osems[k])

        def out_wait(k):
            pltpu.make_async_copy(
                gbufs[k], out_b.at[pl.ds(0, _CHUNK)], osems[k]).wait()

        in_start(0, 0)
        in_start(1, 1)

        def grp(g, carry):
            for k in range(2):
                ch = g * 2 + k
                in_wait(k)
                out_start(ch, k)
                @pl.when(ch + 2 < nch)
                def _():
                    out_wait(k)
                    in_start(ch + 2, k)
            return carry

        lax.fori_loop(0, nch // 2, grp, 0)
        out_wait(0)
        out_wait(1)

    return pl.kernel(
        body,
        out_type=jax.ShapeDtypeStruct((nbh, m, c), jnp.float32),
        mesh=plsc.VectorSubcoreMesh(core_axis_name="c", subcore_axis_name="s"),
        scratch_types=[
            pltpu.VMEM((mper,), jnp.int32),
            pltpu.VMEM((_CHUNK, c), jnp.float32),
            pltpu.VMEM((_CHUNK, c), jnp.float32),
            pltpu.SemaphoreType.DMA,
            pltpu.SemaphoreType.DMA,
            pltpu.SemaphoreType.DMA,
            pltpu.SemaphoreType.DMA,
        ],
        compiler_params=pltpu.CompilerParams(needs_layout_passes=False),
    )


def _tc_transpose(tmp, prev, boff, num_rows, m):
    nbh, _, c = tmp.shape
    grid = (nbh, m // _BM)
    in_spec = pl.BlockSpec((1, _BM, c), lambda bb, mi: (bb, mi, 0))
    out_specs = pl.BlockSpec(
        (c, _BM), lambda bb, mi, boff=boff: (boff + bb, mi))
    out_shape = jax.ShapeDtypeStruct((num_rows, m), jnp.float32)
    if prev is None:
        def tkern0(in_ref, out_ref):
            out_ref[...] = in_ref[0].T

        return pl.pallas_call(
            tkern0, grid=grid, in_specs=[in_spec], out_specs=out_specs,
            out_shape=out_shape)(tmp)

    def tkern(in_ref, prev_ref, out_ref):
        del prev_ref
        out_ref[...] = in_ref[0].T

    return pl.pallas_call(
        tkern, grid=grid,
        in_specs=[in_spec,
                  pl.BlockSpec(memory_space=pltpu.MemorySpace.HBM)],
        out_specs=out_specs, out_shape=out_shape,
        input_output_aliases={1: 0})(tmp, prev)


def kernel(features, idx):
    nb, c, n = features.shape
    m = idx.shape[1]
    ft = jnp.swapaxes(features, 1, 2)
    idx32 = idx.astype(jnp.int32)
    nbh = nb // 2
    gather_lo = _build_sc_gather(0, nbh, n, c, m)
    gather_hi = _build_sc_gather(nbh, nbh, n, c, m)
    tmp_lo = gather_lo(ft, idx32)
    tmp_hi = gather_hi(ft, idx32)
    out1 = _tc_transpose(tmp_lo, None, 0, nb * c, m)
    out2 = _tc_transpose(tmp_hi, out1, nbh, nb * c, m)
    return out2.reshape(nb, c, m)


# speedup vs baseline: 2.1061x; 1.0169x over previous
import jax
import jax.numpy as jnp
from jax import lax
from jax.experimental import pallas as pl
from jax.experimental.pallas import tpu as pltpu
from jax.experimental.pallas import tpu_sc as plsc

_CHUNK = 256
_BM = 2048


def _build_sc_gather(boff, nbh, n, c, m):
    info = plsc.get_sparse_core_info()
    num_workers = info.num_cores * info.num_subcores
    assert num_workers % nbh == 0
    parts = num_workers // nbh
    assert m % parts == 0
    mper = m // parts
    assert mper % _CHUNK == 0
    nch = mper // _CHUNK
    assert nch >= 4 and nch % 2 == 0

    def body(ft_hbm, idx_hbm, out_hbm, idx_v, g_a, g_b, s0, s1, t0, t1):
        gbufs = (g_a, g_b)
        insems = (s0, s1)
        osems = (t0, t1)
        w = lax.axis_index("s") * info.num_cores + lax.axis_index("c")
        b = boff + w // parts
        mb0 = (w % parts) * mper
        pltpu.sync_copy(idx_hbm.at[b].at[pl.ds(mb0, mper)], idx_v)
        table = ft_hbm.at[b]
        out_b = out_hbm.at[b - boff]

        def in_start(ch, k):
            pltpu.async_copy(
                table.at[idx_v.at[pl.ds(ch * _CHUNK, _CHUNK)]],
                gbufs[k], insems[k])

        def in_wait(k):
            pltpu.make_async_copy(
                table.at[idx_v.at[pl.ds(0, _CHUNK)]],
                gbufs[k], insems[k]).wait()

        def out_start(ch, k):
            pltpu.async_copy(
                gbufs[k], out_b.at[pl.ds(mb0 + ch * _CHUNK, _CHUNK)],
                osems[k])

        def out_wait(k):
            pltpu.make_async_copy(
                gbufs[k], out_b.at[pl.ds(0, _CHUNK)], osems[k]).wait()

        in_start(0, 0)
        in_start(1, 1)

        def grp(g, carry):
            for k in range(2):
                ch = g * 2 + k
                in_wait(k)
                out_start(ch, k)
                @pl.when(ch + 2 < nch)
                def _():
                    out_wait(k)
                    in_start(ch + 2, k)
            return carry

        lax.fori_loop(0, nch // 2, grp, 0)
        out_wait(0)
        out_wait(1)

    return pl.kernel(
        body,
        out_type=jax.ShapeDtypeStruct((nbh, m, c), jnp.float32),
        mesh=plsc.VectorSubcoreMesh(core_axis_name="c", subcore_axis_name="s"),
        scratch_types=[
            pltpu.VMEM((mper,), jnp.int32),
            pltpu.VMEM((_CHUNK, c), jnp.float32),
            pltpu.VMEM((_CHUNK, c), jnp.float32),
            pltpu.SemaphoreType.DMA,
            pltpu.SemaphoreType.DMA,
            pltpu.SemaphoreType.DMA,
            pltpu.SemaphoreType.DMA,
        ],
        compiler_params=pltpu.CompilerParams(needs_layout_passes=False),
    )


def _tc_transpose(tmp, prev, boff, num_rows, m):
    nbh, _, c = tmp.shape
    grid = (nbh, m // _BM)
    in_spec = pl.BlockSpec((1, _BM, c), lambda bb, mi: (bb, mi, 0))
    out_specs = pl.BlockSpec(
        (c, _BM), lambda bb, mi, boff=boff: (boff + bb, mi))
    out_shape = jax.ShapeDtypeStruct((num_rows, m), jnp.float32)
    if prev is None:
        def tkern0(in_ref, out_ref):
            out_ref[...] = in_ref[0].T

        return pl.pallas_call(
            tkern0, grid=grid, in_specs=[in_spec], out_specs=out_specs,
            out_shape=out_shape)(tmp)

    def tkern(in_ref, prev_ref, out_ref):
        del prev_ref
        out_ref[...] = in_ref[0].T

    return pl.pallas_call(
        tkern, grid=grid,
        in_specs=[in_spec,
                  pl.BlockSpec(memory_space=pltpu.MemorySpace.HBM)],
        out_specs=out_specs, out_shape=out_shape,
        input_output_aliases={1: 0})(tmp, prev)


def kernel(features, idx):
    nb, c, n = features.shape
    m = idx.shape[1]
    ft = jnp.swapaxes(features, 1, 2)
    idx32 = idx.astype(jnp.int32)
    nparts = 4
    nbh = nb // nparts
    tmps = [
        _build_sc_gather(i * nbh, nbh, n, c, m)(ft, idx32)
        for i in range(nparts)
    ]
    out = None
    for i in range(nparts):
        out = _tc_transpose(tmps[i], out, i * nbh, nb * c, m)
    return out.reshape(nb, c, m)


# speedup vs baseline: 2.4102x; 1.1444x over previous
import jax
import jax.numpy as jnp
from jax import lax
from jax.experimental import pallas as pl
from jax.experimental.pallas import tpu as pltpu
from jax.experimental.pallas import tpu_sc as plsc

_CHUNK = 128
_NBUF = 4


def _build_sc_gather(nb, n, c, m):
    info = plsc.get_sparse_core_info()
    num_workers = info.num_cores * info.num_subcores
    assert num_workers % nb == 0
    parts = num_workers // nb
    assert m % parts == 0
    mper = m // parts
    assert mper % _CHUNK == 0
    nch = mper // _CHUNK
    assert nch % _NBUF == 0 and nch >= 2 * _NBUF

    def body(ft_hbm, idx_hbm, out_hbm, idx_v, g_v, s0, s1, s2, s3,
             t0, t1, t2, t3):
        insems = (s0, s1, s2, s3)
        osems = (t0, t1, t2, t3)
        w = lax.axis_index("s") * info.num_cores + lax.axis_index("c")
        b = w // parts
        mb0 = (w % parts) * mper
        pltpu.sync_copy(idx_hbm.at[b].at[pl.ds(mb0, mper)], idx_v)
        table = ft_hbm.at[b]
        out_b = out_hbm.at[b]

        def in_start(ch, k):
            pltpu.async_copy(
                table.at[idx_v.at[pl.ds(ch * _CHUNK, _CHUNK)]],
                g_v.at[k], insems[k])

        def in_wait(k):
            pltpu.make_async_copy(
                table.at[idx_v.at[pl.ds(0, _CHUNK)]],
                g_v.at[k], insems[k]).wait()

        def out_start(ch, k):
            pltpu.async_copy(
                g_v.at[k], out_b.at[pl.ds(mb0 + ch * _CHUNK, _CHUNK)],
                osems[k])

        def out_wait(k):
            pltpu.make_async_copy(
                g_v.at[k], out_b.at[pl.ds(0, _CHUNK)], osems[k]).wait()

        in_start(0, 0)
        in_start(1, 1)

        def grp(g, carry):
            for k in range(_NBUF):
                ch = g * _NBUF + k
                in_wait(k)
                out_start(ch, k)
                q = (k + 2) % _NBUF
                if k >= 2:
                    @pl.when(ch + 2 < nch)
                    def _():
                        out_wait(q)
                        in_start(ch + 2, q)
                else:
                    @pl.when(g > 0)
                    def _():
                        out_wait(q)
                    in_start(ch + 2, q)
            return carry

        lax.fori_loop(0, nch // _NBUF, grp, 0)
        for k in range(_NBUF):
            out_wait(k)

    return pl.kernel(
        body,
        out_type=jax.ShapeDtypeStruct((nb, m, c), jnp.float32),
        mesh=plsc.VectorSubcoreMesh(core_axis_name="c", subcore_axis_name="s"),
        scratch_types=[
            pltpu.VMEM((m // (num_workers // nb),), jnp.int32),
            pltpu.VMEM((_NBUF, _CHUNK, c), jnp.float32),
            pltpu.SemaphoreType.DMA,
            pltpu.SemaphoreType.DMA,
            pltpu.SemaphoreType.DMA,
            pltpu.SemaphoreType.DMA,
            pltpu.SemaphoreType.DMA,
            pltpu.SemaphoreType.DMA,
            pltpu.SemaphoreType.DMA,
            pltpu.SemaphoreType.DMA,
        ],
        compiler_params=pltpu.CompilerParams(needs_layout_passes=False),
    )


def kernel(features, idx):
    nb, c, n = features.shape
    m = idx.shape[1]
    ft = jnp.swapaxes(features, 1, 2)
    idx32 = idx.astype(jnp.int32)
    gather = _build_sc_gather(nb, n, c, m)
    out3 = gather(ft, idx32)
    return jnp.swapaxes(out3, 1, 2)
